# Initial kernel scaffold; baseline (speedup 1.0000x reference)
import jax
import jax.numpy as jnp
from jax.experimental import pallas as pl


def kernel(edge_idx, node_features, edge_features, e_W0, e_b0, e_W1, e_b1, e_W2, e_b2, e_g, e_beta, n_W0, n_b0, n_W1, n_b1, n_W2, n_b2, n_g, n_beta):
    raise NotImplementedError("write your pallas kernel here")


# speedup vs baseline: 3.6199x; 3.6199x over previous
import functools

import jax
import jax.numpy as jnp
from jax import lax
from jax.experimental import pallas as pl
from jax.experimental.pallas import tpu as pltpu
from jax.experimental.pallas import tpu_sc as plsc

N_NODES = 10000
N_EDGES = 320000
HID = 128

NC = 2
NS = 16
NW = NC * NS
CHUNK = 128
N_CHUNKS = N_EDGES // CHUNK
CHUNKS_PER_CORE = N_CHUNKS // NC
ROWS_PER_TILE = N_NODES // NS


def _k1_body(nf, w0a, w0b, pa, pb):
    x = nf[...]
    pa[...] = jnp.dot(x, w0a[...], preferred_element_type=jnp.float32)
    pb[...] = jnp.dot(x, w0b[...], preferred_element_type=jnp.float32)


def _project(nf, w0a, w0b):
    B = 1000
    grid = N_NODES // B
    return pl.pallas_call(
        _k1_body,
        grid=(grid,),
        in_specs=[
            pl.BlockSpec((B, HID), lambda i: (i, 0)),
            pl.BlockSpec((HID, HID), lambda i: (0, 0)),
            pl.BlockSpec((HID, HID), lambda i: (0, 0)),
        ],
        out_specs=[
            pl.BlockSpec((B, HID), lambda i: (i, 0)),
            pl.BlockSpec((B, HID), lambda i: (i, 0)),
        ],
        out_shape=[
            jax.ShapeDtypeStruct((N_NODES, HID), jnp.float32),
            jax.ShapeDtypeStruct((N_NODES, HID), jnp.float32),
        ],
    )(nf, w0a, w0b)


def _k2_body(sidx_hbm, ridx_hbm, pa_hbm, pb_hbm, ga_hbm, gb_hbm,
             sbuf, rbuf, rows_a, rows_b, sem_a, sem_b):
    wid = lax.axis_index("s") * NC + lax.axis_index("c")

    def chunk(t, carry):
        c = wid + NW * t

        @pl.when(c < N_CHUNKS)
        def _():
            pltpu.sync_copy(sidx_hbm.at[c], sbuf)
            pltpu.sync_copy(ridx_hbm.at[c], rbuf)
            cp_a = pltpu.async_copy(pa_hbm.at[sbuf], rows_a, sem_a)
            cp_b = pltpu.async_copy(pb_hbm.at[rbuf], rows_b, sem_b)
            cp_a.wait()
            cp_b.wait()
            pltpu.sync_copy(rows_a, ga_hbm.at[c])
            pltpu.sync_copy(rows_b, gb_hbm.at[c])

        return carry

    lax.fori_loop(0, (N_CHUNKS + NW - 1) // NW, chunk, 0)


def _gather(sidx, ridx, pa, pb):
    mesh = plsc.VectorSubcoreMesh(core_axis_name="c", subcore_axis_name="s")
    f = pl.kernel(
        _k2_body,
        out_type=[
            jax.ShapeDtypeStruct((N_CHUNKS, CHUNK, HID), jnp.float32),
            jax.ShapeDtypeStruct((N_CHUNKS, CHUNK, HID), jnp.float32),
        ],
        mesh=mesh,
        scratch_types=[
            pltpu.VMEM((CHUNK,), jnp.int32),
            pltpu.VMEM((CHUNK,), jnp.int32),
            pltpu.VMEM((CHUNK, HID), jnp.float32),
            pltpu.VMEM((CHUNK, HID), jnp.float32),
            pltpu.SemaphoreType.DMA,
            pltpu.SemaphoreType.DMA,
        ],
    )
    return f(sidx, ridx, pa, pb)


def _k3_body(ga, gb, ef, w0c, b0, w1, b1, w2, b2, g, beta, pre, res):
    e = ef[...]
    h = ga[...] + gb[...] + jnp.dot(e, w0c[...], preferred_element_type=jnp.float32)
    h = jnp.maximum(h + b0[...], 0.0)
    h = jnp.maximum(jnp.dot(h, w1[...], preferred_element_type=jnp.float32) + b1[...], 0.0)
    h = jnp.dot(h, w2[...], preferred_element_type=jnp.float32) + b2[...]
    mu = jnp.mean(h, axis=-1, keepdims=True)
    d = h - mu
    var = jnp.mean(d * d, axis=-1, keepdims=True)
    ln = g[...] * d * lax.rsqrt(var + 1e-5) + beta[...]
    pre[...] = ln
    res[...] = ln + e


def _edge_mlp(ga, gb, ef, w0c, b0, w1, b1, w2, b2, g, beta):
    B = 2000
    grid = N_EDGES // B
    wspec = pl.BlockSpec((HID, HID), lambda i: (0, 0))
    vspec = pl.BlockSpec((1, HID), lambda i: (0, 0))
    rspec = pl.BlockSpec((B, HID), lambda i: (i, 0))
    return pl.pallas_call(
        _k3_body,
        grid=(grid,),
        in_specs=[rspec, rspec, rspec, wspec, vspec, wspec, vspec, wspec,
                  vspec, vspec, vspec],
        out_specs=[rspec, rspec],
        out_shape=[
            jax.ShapeDtypeStruct((N_EDGES, HID), jnp.float32),
            jax.ShapeDtypeStruct((N_EDGES, HID), jnp.float32),
        ],
    )(ga, gb, ef, w0c, b0, w1, b1, w2, b2, g, beta)


def _k4_body(ridx_hbm, pre_hbm, zeros_hbm, out_hbm, ibuf, rows, acc):
    cid = lax.axis_index("c")
    sid = lax.axis_index("s")

    @pl.when(sid == 0)
    def _():
        pltpu.sync_copy(zeros_hbm, acc)

    plsc.subcore_barrier()

    def chunk(t, carry):
        c = cid * CHUNKS_PER_CORE + sid + NS * t

        @pl.when(sid + NS * t < CHUNKS_PER_CORE)
        def _():
            pltpu.sync_copy(ridx_hbm.at[c], ibuf)
            pltpu.sync_copy(pre_hbm.at[c], rows)
            pltpu.sync_copy(rows, acc.at[ibuf], add=True)

        return carry

    lax.fori_loop(0, (CHUNKS_PER_CORE + NS - 1) // NS, chunk, 0)
    plsc.subcore_barrier()

    @pl.when(sid == 0)
    def _():
        pltpu.sync_copy(acc, out_hbm.at[cid])


def _scatter(ridx, pre, zeros):
    mesh = plsc.VectorSubcoreMesh(core_axis_name="c", subcore_axis_name="s")
    f = pl.kernel(
        _k4_body,
        out_type=jax.ShapeDtypeStruct((NC, N_NODES, HID), jnp.float32),
        mesh=mesh,
        scratch_types=[
            pltpu.VMEM((CHUNK,), jnp.int32),
            pltpu.VMEM((CHUNK, HID), jnp.float32),
            pltpu.VMEM_SHARED((N_NODES, HID), jnp.float32),
        ],
    )
    return f(ridx, pre, zeros)


def _k5_body(nf, p0, p1, w0a, w0b, b0, w1, b1, w2, b2, g, beta, out):
    x = nf[...]
    a = p0[0] + p1[0]
    h = (jnp.dot(x, w0a[...], preferred_element_type=jnp.float32)
         + jnp.dot(a, w0b[...], preferred_element_type=jnp.float32))
    h = jnp.maximum(h + b0[...], 0.0)
    h = jnp.maximum(jnp.dot(h, w1[...], preferred_element_type=jnp.float32) + b1[...], 0.0)
    h = jnp.dot(h, w2[...], preferred_element_type=jnp.float32) + b2[...]
    mu = jnp.mean(h, axis=-1, keepdims=True)
    d = h - mu
    var = jnp.mean(d * d, axis=-1, keepdims=True)
    out[...] = g[...] * d * lax.rsqrt(var + 1e-5) + beta[...] + x


def _node_mlp(nf, partials, w0a, w0b, b0, w1, b1, w2, b2, g, beta):
    B = 1000
    grid = N_NODES // B
    wspec = pl.BlockSpec((HID, HID), lambda i: (0, 0))
    vspec = pl.BlockSpec((1, HID), lambda i: (0, 0))
    rspec = pl.BlockSpec((B, HID), lambda i: (i, 0))
    return pl.pallas_call(
        _k5_body,
        grid=(grid,),
        in_specs=[
            rspec,
            pl.BlockSpec((1, B, HID), lambda i: (0, i, 0)),
            pl.BlockSpec((1, B, HID), lambda i: (1, i, 0)),
            wspec, wspec, vspec, wspec, vspec, wspec, vspec, vspec, vspec,
        ],
        out_specs=rspec,
        out_shape=jax.ShapeDtypeStruct((N_NODES, HID), jnp.float32),
    )(nf, partials, partials, w0a, w0b, b0, w1, b1, w2, b2, g, beta)


def kernel(edge_idx, node_features, edge_features,
           e_W0, e_b0, e_W1, e_b1, e_W2, e_b2, e_g, e_beta,
           n_W0, n_b0, n_W1, n_b1, n_W2, n_b2, n_g, n_beta):
    senders = edge_idx[0].astype(jnp.int32).reshape(N_CHUNKS, CHUNK)
    receivers = edge_idx[1].astype(jnp.int32).reshape(N_CHUNKS, CHUNK)

    e_w0a = e_W0[:HID]
    e_w0b = e_W0[HID:2 * HID]
    e_w0c = e_W0[2 * HID:]
    n_w0a = n_W0[:HID]
    n_w0b = n_W0[HID:]

    r1 = lambda v: v.reshape(1, HID)

    pa, pb = _project(node_features, e_w0a, e_w0b)
    ga, gb = _gather(senders, receivers, pa, pb)
    ga = ga.reshape(N_EDGES, HID)
    gb = gb.reshape(N_EDGES, HID)
    pre, new_edge = _edge_mlp(ga, gb, edge_features, e_w0c, r1(e_b0),
                              e_W1, r1(e_b1), e_W2, r1(e_b2),
                              r1(e_g), r1(e_beta))
    zeros = jnp.zeros((N_NODES, HID), jnp.float32)
    partials = _scatter(receivers, pre.reshape(N_CHUNKS, CHUNK, HID), zeros)
    new_node = _node_mlp(node_features, partials, n_w0a, n_w0b, r1(n_b0),
                         n_W1, r1(n_b1), n_W2, r1(n_b2), r1(n_g), r1(n_beta))
    return (new_node, new_edge)
